# baseline (speedup 1.0000x reference)
import functools

import jax
import jax.numpy as jnp
from jax import lax
from jax.experimental import pallas as pl
from jax.experimental.pallas import tpu as pltpu
from jax.experimental.pallas import tpu_sc as plsc

VOCAB = 1000000
EMBED = 64
B = 16384
L = 50
HID = 50
NCLS = 2

NC = 2
NS = 16
NW = NC * NS
ROWS_W = B // NW
RPC = 2
IDXC = RPC * L
CHUNKS = ROWS_W // RPC
NV = EMBED // 16
NBUF = 4


def _sc_body(text_hbm, table_hbm, out_hbm, idx_v, bufs, out_v, sems):
    wid = lax.axis_index("s") * NC + lax.axis_index("c")
    pltpu.sync_copy(text_hbm.at[wid], idx_v)

    def accumulate(buf, g):
        for r in range(RPC):
            row = g * RPC + r
            for k in range(NV):
                acc = buf[r * L, pl.ds(k * 16, 16)]
                for l in range(1, L):
                    acc = acc + buf[r * L + l, pl.ds(k * 16, 16)]
                out_v[row, pl.ds(k * 16, 16)] = acc * (1.0 / L)

    for b in range(NBUF):
        pltpu.async_copy(table_hbm.at[idx_v.at[b]], bufs[b], sems[b])

    def body(gp, _):
        for b in range(NBUF):
            g = gp * NBUF + b
            pltpu.make_async_copy(table_hbm.at[idx_v.at[g]], bufs[b], sems[b]).wait()
            accumulate(bufs[b], g)
            nxt = g + NBUF

            @pl.when(nxt < CHUNKS)
            def _():
                pltpu.async_copy(table_hbm.at[idx_v.at[nxt]], bufs[b], sems[b])

        return 0

    lax.fori_loop(0, CHUNKS // NBUF, body, 0)

    pltpu.sync_copy(out_v, out_hbm.at[pl.ds(wid * ROWS_W, ROWS_W)])


def _sc_pool(text_r, table):
    mesh = plsc.VectorSubcoreMesh(core_axis_name="c", subcore_axis_name="s")
    return pl.kernel(
        _sc_body,
        out_type=jax.ShapeDtypeStruct((B, EMBED), jnp.float32),
        mesh=mesh,
        scratch_types=[
            pltpu.VMEM((CHUNKS, IDXC), jnp.int32),
            [pltpu.VMEM((IDXC, EMBED), jnp.float32) for _ in range(NBUF)],
            pltpu.VMEM((ROWS_W, EMBED), jnp.float32),
            [pltpu.SemaphoreType.DMA for _ in range(NBUF)],
        ],
        compiler_params=pltpu.CompilerParams(use_tc_tiling_on_sc=False),
    )(text_r, table)


def _mlp_body(feat_ref, w1t_ref, b1_ref, w2t_ref, b2_ref, out_ref):
    h = jnp.dot(feat_ref[...], w1t_ref[...], preferred_element_type=jnp.float32)
    h = jnp.maximum(h + b1_ref[...], 0.0)
    out_ref[...] = jnp.dot(h, w2t_ref[...], preferred_element_type=jnp.float32) + b2_ref[...]


def _mlp(feat, w1t, b1r, w2t, b2r):
    return pl.pallas_call(
        _mlp_body,
        out_shape=jax.ShapeDtypeStruct((B, NCLS), jnp.float32),
    )(feat, w1t, b1r, w2t, b2r)


def kernel(text, text_len, table, W1, b1, W2, b2):
    text_r = text.reshape(NW, CHUNKS, IDXC)
    pooled = _sc_pool(text_r, table)

    len_col = text_len.astype(jnp.float32).reshape(B, 1)
    feat = jnp.concatenate([pooled, len_col], axis=1)
    out = _mlp(feat, W1.T, b1.reshape(1, HID), W2.T, b2.reshape(1, NCLS))
    return out


# speedup vs baseline: 1.2618x; 1.2618x over previous
import functools

import jax
import jax.numpy as jnp
from jax import lax
from jax.experimental import pallas as pl
from jax.experimental.pallas import tpu as pltpu
from jax.experimental.pallas import tpu_sc as plsc

VOCAB = 1000000
EMBED = 64
B = 16384
L = 50
HID = 50
NCLS = 2

NC = 2
NS = 16
NW = NC * NS
ROWS_W = B // NW
NV = EMBED // 16
NBUF = 4


def _sc_body(text_hbm, table_hbm, out_hbm, idx_v, bufs, out_v, sems):
    wid = lax.axis_index("s") * NC + lax.axis_index("c")
    pltpu.sync_copy(text_hbm.at[pl.ds(wid * ROWS_W, ROWS_W)], idx_v)

    def accumulate(buf, b):
        accs = [buf[0, pl.ds(k * 16, 16)] for k in range(NV)]
        for l in range(1, L):
            for k in range(NV):
                accs[k] = accs[k] + buf[l, pl.ds(k * 16, 16)]
        for k in range(NV):
            out_v[b, pl.ds(k * 16, 16)] = accs[k] * (1.0 / L)

    for j in range(NBUF):
        pltpu.async_copy(table_hbm.at[idx_v.at[j]], bufs[j], sems[j])

    def body(gp, _):
        for j in range(NBUF):
            b = gp * NBUF + j
            pltpu.make_async_copy(table_hbm.at[idx_v.at[b]], bufs[j], sems[j]).wait()
            accumulate(bufs[j], b)
            nxt = b + NBUF

            @pl.when(nxt < ROWS_W)
            def _():
                pltpu.async_copy(table_hbm.at[idx_v.at[nxt]], bufs[j], sems[j])

        return 0

    lax.fori_loop(0, ROWS_W // NBUF, body, 0)

    pltpu.sync_copy(out_v, out_hbm.at[pl.ds(wid * ROWS_W, ROWS_W)])


def _sc_pool(text, table):
    mesh = plsc.VectorSubcoreMesh(core_axis_name="c", subcore_axis_name="s")
    return pl.kernel(
        _sc_body,
        out_type=jax.ShapeDtypeStruct((B, EMBED), jnp.float32),
        mesh=mesh,
        scratch_types=[
            pltpu.VMEM((ROWS_W, L), jnp.int32),
            [pltpu.VMEM((L, EMBED), jnp.float32) for _ in range(NBUF)],
            pltpu.VMEM((ROWS_W, EMBED), jnp.float32),
            [pltpu.SemaphoreType.DMA for _ in range(NBUF)],
        ],
        compiler_params=pltpu.CompilerParams(use_tc_tiling_on_sc=False),
    )(text, table)


def _mlp_body(feat_ref, w1t_ref, b1_ref, w2t_ref, b2_ref, out_ref):
    h = jnp.dot(feat_ref[...], w1t_ref[...], preferred_element_type=jnp.float32)
    h = jnp.maximum(h + b1_ref[...], 0.0)
    out_ref[...] = jnp.dot(h, w2t_ref[...], preferred_element_type=jnp.float32) + b2_ref[...]


def _mlp(feat, w1t, b1r, w2t, b2r):
    return pl.pallas_call(
        _mlp_body,
        out_shape=jax.ShapeDtypeStruct((B, NCLS), jnp.float32),
    )(feat, w1t, b1r, w2t, b2r)


def kernel(text, text_len, table, W1, b1, W2, b2):
    pooled = _sc_pool(text, table)

    len_col = text_len.astype(jnp.float32).reshape(B, 1)
    feat = jnp.concatenate([pooled, len_col], axis=1)
    out = _mlp(feat, W1.T, b1.reshape(1, HID), W2.T, b2.reshape(1, NCLS))
    return out
